# initial kernel scaffold (unmeasured)
import jax
import jax.numpy as jnp
from jax import lax
from jax.experimental import pallas as pl
from jax.experimental.pallas import tpu as pltpu

N_DEV = 8
SQ = 256
SKV = 4096
HQ = 8
DH = 128
DM = 1024
SCALE = 0.08838834764831843


def kernel(x, Wq, K_ext, V_ext, Wo):
    def body(x_ref, wq_ref, k_ref, v_ref, wo_ref, out_ref,
             o_comm, l_comm, o_send, o_recv, l_send, l_recv):
        my = lax.axis_index("i")
        left = lax.rem(my + N_DEV - 1, N_DEV)
        right = lax.rem(my + 1, N_DEV)

        barrier_sem = pltpu.get_barrier_semaphore()
        for nbr in (left, right):
            pl.semaphore_signal(
                barrier_sem, inc=1,
                device_id=(nbr,), device_id_type=pl.DeviceIdType.MESH,
            )
        pl.semaphore_wait(barrier_sem, 2)

        xb = x_ref[0, :, :].astype(jnp.bfloat16)
        wq = wq_ref[:, :].astype(jnp.bfloat16)
        q = jnp.dot(xb, wq, preferred_element_type=jnp.float32)
        qs = (q * SCALE).reshape(SQ, HQ, DH).astype(jnp.bfloat16)

        qi = lax.broadcasted_iota(jnp.int32, (SQ, SKV), 0)
        ki = lax.broadcasted_iota(jnp.int32, (SQ, SKV), 1)
        mask = ((ki // 64) % 4) == (qi // 64)

        for h in range(HQ):
            qh = qs[:, h, :]
            kh = k_ref[0, :, h, :].astype(jnp.bfloat16)
            s = lax.dot_general(
                qh, kh, (((1,), (1,)), ((), ())),
                preferred_element_type=jnp.float32,
            )
            e = jnp.exp(jnp.where(mask, s, -1e30))
            l_comm[0, h, :] = e.sum(axis=1)
            vh = v_ref[0, :, h, :].astype(jnp.bfloat16)
            o = lax.dot_general(
                e.astype(jnp.bfloat16), vh, (((1,), (0,)), ((), ())),
                preferred_element_type=jnp.float32,
            )
            o_comm[0, h, :, :] = o.astype(jnp.bfloat16)

        for h in range(N_DEV - 1):
            rdma_o = pltpu.make_async_remote_copy(
                src_ref=o_comm.at[h], dst_ref=o_comm.at[h + 1],
                send_sem=o_send.at[h], recv_sem=o_recv.at[h],
                device_id=(right,), device_id_type=pl.DeviceIdType.MESH,
            )
            rdma_l = pltpu.make_async_remote_copy(
                src_ref=l_comm.at[h], dst_ref=l_comm.at[h + 1],
                send_sem=l_send.at[h], recv_sem=l_recv.at[h],
                device_id=(right,), device_id_type=pl.DeviceIdType.MESH,
            )
            rdma_o.start()
            rdma_l.start()
            rdma_o.wait()
            rdma_l.wait()

        o_sum = o_comm[:, :, :, :].astype(jnp.float32).sum(axis=0)
        l_sum = l_comm[:, :, :].sum(axis=0)
        ctx = o_sum / l_sum[:, :, None]
        ctx = ctx.transpose(1, 0, 2).reshape(SQ, DM).astype(jnp.bfloat16)
        wo = wo_ref[:, :].astype(jnp.bfloat16)
        out_ref[0, :, :] = jnp.dot(ctx, wo, preferred_element_type=jnp.float32)

    return pl.pallas_call(
        body,
        out_shape=jax.ShapeDtypeStruct((1, SQ, DM), jnp.float32),
        in_specs=[pl.BlockSpec(memory_space=pltpu.VMEM)] * 5,
        out_specs=pl.BlockSpec(memory_space=pltpu.VMEM),
        scratch_shapes=[
            pltpu.VMEM((N_DEV, HQ, SQ, DH), jnp.bfloat16),
            pltpu.VMEM((N_DEV, HQ, SQ), jnp.float32),
            pltpu.SemaphoreType.DMA((N_DEV - 1,)),
            pltpu.SemaphoreType.DMA((N_DEV - 1,)),
            pltpu.SemaphoreType.DMA((N_DEV - 1,)),
            pltpu.SemaphoreType.DMA((N_DEV - 1,)),
        ],
        compiler_params=pltpu.CompilerParams(collective_id=0),
    )(x, Wq, K_ext, V_ext, Wo)


# baseline (device time: 127612 ns/iter reference)
import jax
import jax.numpy as jnp
from jax import lax
from jax.experimental import pallas as pl
from jax.experimental.pallas import tpu as pltpu

N_DEV = 8
SQ = 256
SKV = 4096
HQ = 8
DH = 128
DM = 1024
SCALE = 0.08838834764831843


def kernel(x, Wq, K_ext, V_ext, Wo):
    def body(x_ref, wq_ref, k_ref, v_ref, wo_ref, out_ref,
             o_comm, l_comm, o_send, o_recv, l_send, l_recv):
        my = lax.axis_index("i")
        left = lax.rem(my + N_DEV - 1, N_DEV)
        right = lax.rem(my + 1, N_DEV)

        barrier_sem = pltpu.get_barrier_semaphore()
        for nbr in (left, right):
            pl.semaphore_signal(
                barrier_sem, inc=1,
                device_id=(nbr,), device_id_type=pl.DeviceIdType.MESH,
            )
        pl.semaphore_wait(barrier_sem, 2)

        xb = x_ref[0, :, :].astype(jnp.bfloat16)
        wq = wq_ref[:, :].astype(jnp.bfloat16)
        q = jnp.dot(xb, wq, preferred_element_type=jnp.float32)
        qs = (q * SCALE).reshape(SQ, HQ, DH).astype(jnp.bfloat16)

        qi = lax.broadcasted_iota(jnp.int32, (SQ, SKV), 0)
        ki = lax.broadcasted_iota(jnp.int32, (SQ, SKV), 1)
        mask = ((ki // 64) % 4) == (qi // 64)

        for h in range(HQ):
            qh = qs[:, h, :]
            kh = k_ref[0, :, h, :].astype(jnp.bfloat16)
            s = lax.dot_general(
                qh, kh, (((1,), (1,)), ((), ())),
                preferred_element_type=jnp.float32,
            )
            e = jnp.exp(jnp.where(mask, s, -1e30))
            l_comm[0, h, :] = e.sum(axis=1)
            vh = v_ref[0, :, h, :].astype(jnp.bfloat16)
            o = lax.dot_general(
                e.astype(jnp.bfloat16), vh, (((1,), (0,)), ((), ())),
                preferred_element_type=jnp.float32,
            )
            o_comm[0, h, :, :] = o.astype(jnp.bfloat16)

        for h in range(N_DEV - 1):
            rdma_o = pltpu.make_async_remote_copy(
                src_ref=o_comm.at[h], dst_ref=o_comm.at[h + 1],
                send_sem=o_send.at[h], recv_sem=o_recv.at[h],
                device_id=(right,), device_id_type=pl.DeviceIdType.MESH,
            )
            rdma_l = pltpu.make_async_remote_copy(
                src_ref=l_comm.at[h], dst_ref=l_comm.at[h + 1],
                send_sem=l_send.at[h], recv_sem=l_recv.at[h],
                device_id=(right,), device_id_type=pl.DeviceIdType.MESH,
            )
            rdma_o.start()
            rdma_l.start()
            rdma_o.wait()
            rdma_l.wait()

        o_sum = o_comm[:, :, :, :].astype(jnp.float32).sum(axis=0)
        l_sum = l_comm[:, :, :].sum(axis=0)
        ctx = o_sum / l_sum[:, :, None]
        ctx = ctx.transpose(1, 0, 2).reshape(SQ, DM).astype(jnp.bfloat16)
        wo = wo_ref[:, :].astype(jnp.bfloat16)
        out_ref[0, :, :] = jnp.dot(ctx, wo, preferred_element_type=jnp.float32)

    return pl.pallas_call(
        body,
        out_shape=jax.ShapeDtypeStruct((1, SQ, DM), jnp.float32),
        in_specs=[pl.BlockSpec(memory_space=pltpu.VMEM)] * 5,
        out_specs=pl.BlockSpec(memory_space=pltpu.VMEM),
        scratch_shapes=[
            pltpu.VMEM((N_DEV, HQ, SQ, DH), jnp.bfloat16),
            pltpu.VMEM((N_DEV, HQ, SQ), jnp.float32),
            pltpu.SemaphoreType.DMA((N_DEV - 1,)),
            pltpu.SemaphoreType.DMA((N_DEV - 1,)),
            pltpu.SemaphoreType.DMA((N_DEV - 1,)),
            pltpu.SemaphoreType.DMA((N_DEV - 1,)),
        ],
        compiler_params=pltpu.CompilerParams(
            collective_id=0, vmem_limit_bytes=64 * 1024 * 1024
        ),
    )(x, Wq, K_ext, V_ext, Wo)


# device time: 97811 ns/iter; 1.3047x vs baseline; 1.3047x over previous
import jax
import jax.numpy as jnp
from jax import lax
from jax.experimental import pallas as pl
from jax.experimental.pallas import tpu as pltpu

N_DEV = 8
SQ = 256
SKV = 4096
HQ = 8
DH = 128
DM = 1024
SCALE = 0.08838834764831843


def kernel(x, Wq, K_ext, V_ext, Wo):
    def body(x_ref, wq_ref, k_ref, v_ref, wo_ref, out_ref,
             o_send, o_rx, l_send, l_rx, o_ssem, o_rsem, l_ssem, l_rsem):
        my = lax.axis_index("i")

        barrier_sem = pltpu.get_barrier_semaphore()
        for v in (1, 2, 4):
            pl.semaphore_signal(
                barrier_sem, inc=1,
                device_id=(my ^ v,), device_id_type=pl.DeviceIdType.MESH,
            )
        pl.semaphore_wait(barrier_sem, 3)

        xb = x_ref[0, :, :].astype(jnp.bfloat16)
        wq = wq_ref[:, :].astype(jnp.bfloat16)
        q = jnp.dot(xb, wq, preferred_element_type=jnp.float32)
        qs = (q * SCALE).reshape(SQ, HQ, DH).astype(jnp.bfloat16)

        qi = lax.broadcasted_iota(jnp.int32, (SQ, SKV), 0)
        ki = lax.broadcasted_iota(jnp.int32, (SQ, SKV), 1)
        mask = ((ki // 64) % 4) == (qi // 64)

        o_parts = []
        l_parts = []
        for h in range(HQ):
            qh = qs[:, h, :]
            kh = k_ref[0, :, h, :].astype(jnp.bfloat16)
            s = lax.dot_general(
                qh, kh, (((1,), (1,)), ((), ())),
                preferred_element_type=jnp.float32,
            )
            e = jnp.exp(jnp.where(mask, s, -1e30))
            l_parts.append(e.sum(axis=1))
            vh = v_ref[0, :, h, :].astype(jnp.bfloat16)
            o_parts.append(lax.dot_general(
                e.astype(jnp.bfloat16), vh, (((1,), (0,)), ((), ())),
                preferred_element_type=jnp.float32,
            ))

        acc_o = jnp.concatenate(o_parts, axis=1)
        acc_l = jnp.stack(l_parts, axis=0)

        for s, v in enumerate((1, 2, 4)):
            p = my ^ v
            o_send[:, :] = acc_o.astype(jnp.bfloat16)
            l_send[:, :] = acc_l
            rdma_o = pltpu.make_async_remote_copy(
                src_ref=o_send, dst_ref=o_rx.at[s],
                send_sem=o_ssem.at[s], recv_sem=o_rsem.at[s],
                device_id=(p,), device_id_type=pl.DeviceIdType.MESH,
            )
            rdma_l = pltpu.make_async_remote_copy(
                src_ref=l_send, dst_ref=l_rx.at[s],
                send_sem=l_ssem.at[s], recv_sem=l_rsem.at[s],
                device_id=(p,), device_id_type=pl.DeviceIdType.MESH,
            )
            rdma_o.start()
            rdma_l.start()
            rdma_o.wait()
            rdma_l.wait()
            acc_o = acc_o + o_rx[s, :, :].astype(jnp.float32)
            acc_l = acc_l + l_rx[s, :, :]

        l_q = acc_l.transpose(1, 0)
        l_b = jnp.broadcast_to(l_q[:, :, None], (SQ, HQ, DH)).reshape(SQ, DM)
        ctx = (acc_o / l_b).astype(jnp.bfloat16)
        wo = wo_ref[:, :].astype(jnp.bfloat16)
        out_ref[0, :, :] = jnp.dot(ctx, wo, preferred_element_type=jnp.float32)

    return pl.pallas_call(
        body,
        out_shape=jax.ShapeDtypeStruct((1, SQ, DM), jnp.float32),
        in_specs=[pl.BlockSpec(memory_space=pltpu.VMEM)] * 5,
        out_specs=pl.BlockSpec(memory_space=pltpu.VMEM),
        scratch_shapes=[
            pltpu.VMEM((SQ, DM), jnp.bfloat16),
            pltpu.VMEM((3, SQ, DM), jnp.bfloat16),
            pltpu.VMEM((HQ, SQ), jnp.float32),
            pltpu.VMEM((3, HQ, SQ), jnp.float32),
            pltpu.SemaphoreType.DMA((3,)),
            pltpu.SemaphoreType.DMA((3,)),
            pltpu.SemaphoreType.DMA((3,)),
            pltpu.SemaphoreType.DMA((3,)),
        ],
        compiler_params=pltpu.CompilerParams(
            collective_id=0, vmem_limit_bytes=64 * 1024 * 1024
        ),
    )(x, Wq, K_ext, V_ext, Wo)
